# baseline (speedup 1.0000x reference)
import jax
import jax.numpy as jnp
from jax.experimental import pallas as pl
from jax.experimental.pallas import tpu as pltpu

B = 64
N = 128
F = 512
H1 = 256
H2 = 128
EMB = 32
THR = 0.62

TB = 16
KB = 512
NK = N * H2 // KB
NBUF = 4
NPB = KB // H2


def _fused_kernel(adj_ref, x_hbm, w1_ref, b1_ref, w2_ref, b2_ref, wp_hbm,
                  bp_ref, out_ref, x_ref, h_ref, acc_ref, wbuf_ref,
                  wp_sem, x_sem):
    for i in range(NBUF):
        pltpu.make_async_copy(wp_hbm.at[pl.ds(i * KB, KB), :],
                              wbuf_ref.at[i], wp_sem.at[i]).start()
    x_copy = pltpu.make_async_copy(x_hbm, x_ref, x_sem)
    x_copy.start()

    a = jax.nn.sigmoid(adj_ref[...])
    rows = jax.lax.broadcasted_iota(jnp.int32, (N, N), 0)
    cols = jax.lax.broadcasted_iota(jnp.int32, (N, N), 1)
    w = (a > THR).astype(jnp.float32) + (rows == cols).astype(jnp.float32)
    deg = jnp.sum(w, axis=0)
    dinv = jax.lax.rsqrt(deg)
    m = w.T * (dinv[:, None] * dinv[None, :])
    mb = jnp.broadcast_to(m, (TB, N, N))

    x_copy.wait()
    for i in range(B // TB):
        xb = x_ref[i * TB:(i + 1) * TB]
        t1 = jax.lax.dot_general(xb, w1_ref[...], (((2,), (0,)), ((), ())),
                                 preferred_element_type=jnp.float32)
        agg1 = jax.lax.dot_general(mb, t1, (((2,), (1,)), ((0,), (0,))),
                                   preferred_element_type=jnp.float32)
        h1 = jnp.maximum(agg1 + b1_ref[0], 0.0)
        t2 = jax.lax.dot_general(h1, w2_ref[...], (((2,), (0,)), ((), ())),
                                 preferred_element_type=jnp.float32)
        agg2 = jax.lax.dot_general(mb, t2, (((2,), (1,)), ((0,), (0,))),
                                   preferred_element_type=jnp.float32)
        h_ref[i * TB:(i + 1) * TB] = jnp.maximum(agg2 + b2_ref[0], 0.0)

    acc_ref[...] = jnp.zeros((B, N * EMB), jnp.float32)

    def body(k, carry):
        buf = jax.lax.rem(k, NBUF)
        pltpu.make_async_copy(wp_hbm.at[pl.ds(k * KB, KB), :],
                              wbuf_ref.at[buf], wp_sem.at[buf]).wait()
        wblk = wbuf_ref[buf]
        hblk = h_ref[:, pl.ds(k * NPB, NPB), :]
        part = acc_ref[...]
        for c in range(NPB):
            part = part + jnp.dot(hblk[:, c, :],
                                  wblk[c * H2:(c + 1) * H2, :],
                                  preferred_element_type=jnp.float32)
        acc_ref[...] = part

        @pl.when(k + NBUF < NK)
        def _():
            pltpu.make_async_copy(wp_hbm.at[pl.ds((k + NBUF) * KB, KB), :],
                                  wbuf_ref.at[buf], wp_sem.at[buf]).start()
        return carry

    jax.lax.fori_loop(0, NK, body, 0)
    out_ref[...] = jnp.tanh(acc_ref[...] + bp_ref[...])


def kernel(x, adj, W1, b1, W2, b2, Wp, bp):
    out = pl.pallas_call(
        _fused_kernel,
        in_specs=[
            pl.BlockSpec(memory_space=pltpu.MemorySpace.VMEM),
            pl.BlockSpec(memory_space=pl.ANY),
            pl.BlockSpec(memory_space=pltpu.MemorySpace.VMEM),
            pl.BlockSpec(memory_space=pltpu.MemorySpace.VMEM),
            pl.BlockSpec(memory_space=pltpu.MemorySpace.VMEM),
            pl.BlockSpec(memory_space=pltpu.MemorySpace.VMEM),
            pl.BlockSpec(memory_space=pl.ANY),
            pl.BlockSpec(memory_space=pltpu.MemorySpace.VMEM),
        ],
        out_specs=pl.BlockSpec(memory_space=pltpu.MemorySpace.VMEM),
        out_shape=jax.ShapeDtypeStruct((B, N * EMB), jnp.float32),
        scratch_shapes=[
            pltpu.VMEM((B, N, F), jnp.float32),
            pltpu.VMEM((B, N, H2), jnp.float32),
            pltpu.VMEM((B, N * EMB), jnp.float32),
            pltpu.VMEM((NBUF, KB, N * EMB), jnp.float32),
            pltpu.SemaphoreType.DMA((NBUF,)),
            pltpu.SemaphoreType.DMA,
        ],
    )(adj, x, W1, b1.reshape(1, H1), W2, b2.reshape(1, H2), Wp,
      bp.reshape(1, N * EMB))
    return out.reshape(B, N, EMB)


# speedup vs baseline: 1.0036x; 1.0036x over previous
import jax
import jax.numpy as jnp
from jax.experimental import pallas as pl
from jax.experimental.pallas import tpu as pltpu

B = 64
N = 128
F = 512
H1 = 256
H2 = 128
EMB = 32
THR = 0.62

TB = 16
KB = 256
NK = N * H2 // KB
NBUF = 8
NPB = KB // H2


def _fused_kernel(adj_ref, x_hbm, w1_ref, b1_ref, w2_ref, b2_ref, wp_hbm,
                  bp_ref, out_ref, x_ref, h_ref, acc_ref, wbuf_ref,
                  wp_sem, x_sem):
    for i in range(NBUF):
        pltpu.make_async_copy(wp_hbm.at[pl.ds(i * KB, KB), :],
                              wbuf_ref.at[i], wp_sem.at[i]).start()
    x_copy = pltpu.make_async_copy(x_hbm, x_ref, x_sem)
    x_copy.start()

    a = jax.nn.sigmoid(adj_ref[...])
    rows = jax.lax.broadcasted_iota(jnp.int32, (N, N), 0)
    cols = jax.lax.broadcasted_iota(jnp.int32, (N, N), 1)
    w = (a > THR).astype(jnp.float32) + (rows == cols).astype(jnp.float32)
    deg = jnp.sum(w, axis=0)
    dinv = jax.lax.rsqrt(deg)
    m = w.T * (dinv[:, None] * dinv[None, :])
    mb = jnp.broadcast_to(m, (TB, N, N))

    x_copy.wait()
    for i in range(B // TB):
        xb = x_ref[i * TB:(i + 1) * TB]
        t1 = jax.lax.dot_general(xb, w1_ref[...], (((2,), (0,)), ((), ())),
                                 preferred_element_type=jnp.float32)
        agg1 = jax.lax.dot_general(mb, t1, (((2,), (1,)), ((0,), (0,))),
                                   preferred_element_type=jnp.float32)
        h1 = jnp.maximum(agg1 + b1_ref[0], 0.0)
        t2 = jax.lax.dot_general(h1, w2_ref[...], (((2,), (0,)), ((), ())),
                                 preferred_element_type=jnp.float32)
        agg2 = jax.lax.dot_general(mb, t2, (((2,), (1,)), ((0,), (0,))),
                                   preferred_element_type=jnp.float32)
        h_ref[i * TB:(i + 1) * TB] = jnp.maximum(agg2 + b2_ref[0], 0.0)

    acc_ref[...] = jnp.zeros((B, N * EMB), jnp.float32)

    def body(k, carry):
        buf = jax.lax.rem(k, NBUF)
        pltpu.make_async_copy(wp_hbm.at[pl.ds(k * KB, KB), :],
                              wbuf_ref.at[buf], wp_sem.at[buf]).wait()
        wblk = wbuf_ref[buf]
        hblk = h_ref[:, pl.ds(k * NPB, NPB), :]
        part = acc_ref[...]
        for c in range(NPB):
            part = part + jnp.dot(hblk[:, c, :],
                                  wblk[c * H2:(c + 1) * H2, :],
                                  preferred_element_type=jnp.float32)
        acc_ref[...] = part

        @pl.when(k + NBUF < NK)
        def _():
            pltpu.make_async_copy(wp_hbm.at[pl.ds((k + NBUF) * KB, KB), :],
                                  wbuf_ref.at[buf], wp_sem.at[buf]).start()
        return carry

    jax.lax.fori_loop(0, NK, body, 0)
    out_ref[...] = jnp.tanh(acc_ref[...] + bp_ref[...])


def kernel(x, adj, W1, b1, W2, b2, Wp, bp):
    out = pl.pallas_call(
        _fused_kernel,
        in_specs=[
            pl.BlockSpec(memory_space=pltpu.MemorySpace.VMEM),
            pl.BlockSpec(memory_space=pl.ANY),
            pl.BlockSpec(memory_space=pltpu.MemorySpace.VMEM),
            pl.BlockSpec(memory_space=pltpu.MemorySpace.VMEM),
            pl.BlockSpec(memory_space=pltpu.MemorySpace.VMEM),
            pl.BlockSpec(memory_space=pltpu.MemorySpace.VMEM),
            pl.BlockSpec(memory_space=pl.ANY),
            pl.BlockSpec(memory_space=pltpu.MemorySpace.VMEM),
        ],
        out_specs=pl.BlockSpec(memory_space=pltpu.MemorySpace.VMEM),
        out_shape=jax.ShapeDtypeStruct((B, N * EMB), jnp.float32),
        scratch_shapes=[
            pltpu.VMEM((B, N, F), jnp.float32),
            pltpu.VMEM((B, N, H2), jnp.float32),
            pltpu.VMEM((B, N * EMB), jnp.float32),
            pltpu.VMEM((NBUF, KB, N * EMB), jnp.float32),
            pltpu.SemaphoreType.DMA((NBUF,)),
            pltpu.SemaphoreType.DMA,
        ],
    )(adj, x, W1, b1.reshape(1, H1), W2, b2.reshape(1, H2), Wp,
      bp.reshape(1, N * EMB))
    return out.reshape(B, N, EMB)


# speedup vs baseline: 1.0221x; 1.0184x over previous
import jax
import jax.numpy as jnp
from jax.experimental import pallas as pl
from jax.experimental.pallas import tpu as pltpu

B = 64
N = 128
F = 512
H1 = 256
H2 = 128
EMB = 32
THR = 0.62

TB = 8
NXT = B // TB
KB = 512
NK = N * H2 // KB
NBUF = 5
NPB = KB // H2


def _fused_kernel(adj_ref, x_hbm, w1_ref, b1_ref, w2_ref, b2_ref, wp_hbm,
                  bp_ref, out_ref, xbuf_ref, h_ref, acc_ref, wbuf_ref,
                  wp_sem, x_sem):
    for i in range(NBUF):
        pltpu.make_async_copy(wp_hbm.at[pl.ds(i * KB, KB), :],
                              wbuf_ref.at[i], wp_sem.at[i]).start()
    for i in range(2):
        pltpu.make_async_copy(x_hbm.at[pl.ds(i * TB, TB), :, :],
                              xbuf_ref.at[i], x_sem.at[i]).start()

    a = jax.nn.sigmoid(adj_ref[...])
    rows = jax.lax.broadcasted_iota(jnp.int32, (N, N), 0)
    cols = jax.lax.broadcasted_iota(jnp.int32, (N, N), 1)
    w = (a > THR).astype(jnp.float32) + (rows == cols).astype(jnp.float32)
    deg = jnp.sum(w, axis=0)
    dinv = jax.lax.rsqrt(deg)
    m = w.T * (dinv[:, None] * dinv[None, :])
    mb = jnp.broadcast_to(m, (TB, N, N))

    for i in range(NXT):
        slot = i % 2
        pltpu.make_async_copy(x_hbm.at[pl.ds(i * TB, TB), :, :],
                              xbuf_ref.at[slot], x_sem.at[slot]).wait()
        xb = xbuf_ref[slot]
        t1 = jax.lax.dot_general(xb, w1_ref[...], (((2,), (0,)), ((), ())),
                                 preferred_element_type=jnp.float32)
        agg1 = jax.lax.dot_general(mb, t1, (((2,), (1,)), ((0,), (0,))),
                                   preferred_element_type=jnp.float32)
        h1 = jnp.maximum(agg1 + b1_ref[0], 0.0)
        t2 = jax.lax.dot_general(h1, w2_ref[...], (((2,), (0,)), ((), ())),
                                 preferred_element_type=jnp.float32)
        agg2 = jax.lax.dot_general(mb, t2, (((2,), (1,)), ((0,), (0,))),
                                   preferred_element_type=jnp.float32)
        h_ref[i * TB:(i + 1) * TB] = jnp.maximum(agg2 + b2_ref[0], 0.0)
        if i + 2 < NXT:
            pltpu.make_async_copy(x_hbm.at[pl.ds((i + 2) * TB, TB), :, :],
                                  xbuf_ref.at[slot], x_sem.at[slot]).start()

    acc_ref[...] = jnp.zeros((B, N * EMB), jnp.float32)

    def body(k, carry):
        buf = jax.lax.rem(k, NBUF)
        pltpu.make_async_copy(wp_hbm.at[pl.ds(k * KB, KB), :],
                              wbuf_ref.at[buf], wp_sem.at[buf]).wait()
        wblk = wbuf_ref[buf]
        hblk = h_ref[:, pl.ds(k * NPB, NPB), :]
        part = acc_ref[...]
        for c in range(NPB):
            part = part + jnp.dot(hblk[:, c, :],
                                  wblk[c * H2:(c + 1) * H2, :],
                                  preferred_element_type=jnp.float32)
        acc_ref[...] = part

        @pl.when(k + NBUF < NK)
        def _():
            pltpu.make_async_copy(wp_hbm.at[pl.ds((k + NBUF) * KB, KB), :],
                                  wbuf_ref.at[buf], wp_sem.at[buf]).start()
        return carry

    jax.lax.fori_loop(0, NK, body, 0)
    out_ref[...] = jnp.tanh(acc_ref[...] + bp_ref[...])


def kernel(x, adj, W1, b1, W2, b2, Wp, bp):
    out = pl.pallas_call(
        _fused_kernel,
        in_specs=[
            pl.BlockSpec(memory_space=pltpu.MemorySpace.VMEM),
            pl.BlockSpec(memory_space=pl.ANY),
            pl.BlockSpec(memory_space=pltpu.MemorySpace.VMEM),
            pl.BlockSpec(memory_space=pltpu.MemorySpace.VMEM),
            pl.BlockSpec(memory_space=pltpu.MemorySpace.VMEM),
            pl.BlockSpec(memory_space=pltpu.MemorySpace.VMEM),
            pl.BlockSpec(memory_space=pl.ANY),
            pl.BlockSpec(memory_space=pltpu.MemorySpace.VMEM),
        ],
        out_specs=pl.BlockSpec(memory_space=pltpu.MemorySpace.VMEM),
        out_shape=jax.ShapeDtypeStruct((B, N * EMB), jnp.float32),
        scratch_shapes=[
            pltpu.VMEM((2, TB, N, F), jnp.float32),
            pltpu.VMEM((B, N, H2), jnp.float32),
            pltpu.VMEM((B, N * EMB), jnp.float32),
            pltpu.VMEM((NBUF, KB, N * EMB), jnp.float32),
            pltpu.SemaphoreType.DMA((NBUF,)),
            pltpu.SemaphoreType.DMA((2,)),
        ],
    )(adj, x, W1, b1.reshape(1, H1), W2, b2.reshape(1, H2), Wp,
      bp.reshape(1, N * EMB))
    return out.reshape(B, N, EMB)
